# baseline (speedup 1.0000x reference)
import jax
import jax.numpy as jnp
from jax import lax
from jax.experimental import pallas as pl
from jax.experimental.pallas import tpu as pltpu
from jax.experimental.pallas import tpu_sc as plsc

F = 26
D = 16
B = 16384
FIELD = 100000
NPAIR = F * (F - 1) // 2

NC, NS, LANES = 2, 16, 16
NW = NC * NS
WROWS = B // NW
CHUNK = 128
NCHUNK = WROWS // CHUNK
GROUPS = CHUNK // LANES
IW = 128
IROWS = CHUNK * F // IW

_MAGIC = 0x5F3759DF


def _rsqrt(a):
    i = lax.bitcast_convert_type(a, jnp.int32)
    y = lax.bitcast_convert_type(_MAGIC - lax.shift_right_logical(i, 1),
                                 jnp.float32)
    ha = 0.5 * a
    for _ in range(3):
        y = y * (1.5 - ha * y * y)
    return y


def _body(idx_hbm, table_hbm, out_hbm, idx_v, rows_v, out_v, sem):
    c = lax.axis_index("c")
    s = lax.axis_index("s")
    wid = s * NC + c
    base = wid * WROWS
    lane = lax.iota(jnp.int32, LANES)
    rowstep = lane * F

    for ck in range(NCHUNK):
        pltpu.sync_copy(
            idx_hbm.at[pl.ds((base + ck * CHUNK) * F // IW, IROWS)], idx_v)
        descs = [pltpu.async_copy(table_hbm.at[idx_v.at[j]],
                                  rows_v.at[pl.ds(j * IW, IW)], sem)
                 for j in range(IROWS)]
        for dsc in descs:
            dsc.wait()

        def group(g, _):
            rbase = rowstep + g * (LANES * F)

            def fstep(f, carry):
                acc_s, acc_r2, acc_g2 = carry[:3]
                G = carry[3:]
                ridx = rbase + f
                e = [plsc.load_gather(
                        rows_v, [ridx, jnp.full((LANES,), d, jnp.int32)])
                     for d in range(D)]
                n = e[0] * e[0]
                for d in range(1, D):
                    n = n + e[d] * e[d]
                r = _rsqrt(n + 1.0)
                r2 = r * r
                G = tuple(G[d] + r * e[d] for d in range(D))
                return (acc_s + r, acc_r2 + r2, acc_g2 + r2 * n) + G

            z = jnp.zeros((LANES,), jnp.float32)
            res = lax.fori_loop(0, F, fstep, (z,) * (3 + D))
            acc_s, acc_r2, acc_g2 = res[:3]
            G = res[3:]
            gg = G[0] * G[0]
            for d in range(1, D):
                gg = gg + G[d] * G[d]
            logit = (float(NPAIR) + 0.5 * (acc_s * acc_s - acc_r2)
                     - 0.5 * (gg - acc_g2) - float(F - 1) * acc_s)
            out_v[pl.ds(ck * CHUNK + g * LANES, LANES)] = (
                1.0 / (1.0 + jnp.exp(-logit)))
            return 0

        lax.fori_loop(0, GROUPS, group, 0)

    pltpu.sync_copy(out_v, out_hbm.at[pl.ds(base, WROWS)])


_sc_call = pl.kernel(
    _body,
    out_type=jax.ShapeDtypeStruct((B,), jnp.float32),
    mesh=plsc.VectorSubcoreMesh(core_axis_name="c", subcore_axis_name="s"),
    scratch_types=[
        pltpu.VMEM((IROWS, IW), jnp.int32),
        pltpu.VMEM((CHUNK * F, D), jnp.float32),
        pltpu.VMEM((WROWS,), jnp.float32),
        pltpu.SemaphoreType.DMA,
    ],
    compiler_params=pltpu.CompilerParams(use_tc_tiling_on_sc=False,
                                         needs_layout_passes=False),
)


def kernel(x, table):
    offsets = jnp.arange(F, dtype=jnp.int32) * FIELD
    idx = (x + offsets[None, :]).reshape(B * F // IW, IW)
    return _sc_call(idx, table).reshape(B, 1)
